# baseline (speedup 1.0000x reference)
import functools

import jax
import jax.numpy as jnp
from jax import lax
from jax.experimental import pallas as pl
from jax.experimental.pallas import tpu as pltpu
from jax.experimental.pallas import tpu_sc as plsc

D = 64
B = 16384
NC = 2
NS = 16
NW = NC * NS
B_PER_W = B // NW
CHUNK = 128
NCHUNK = B_PER_W // CHUNK
L = 16
BLK = CHUNK // L


def _sc_kernel(i_idx_hbm, j_idx_hbm, wi_hbm, wj_hbm, out_hbm,
               idx_i_v, idx_j_v, rows_i, rows_j, out_v, sem0, sem1):
    wid = lax.axis_index("s") * NC + lax.axis_index("c")
    base = wid * B_PER_W
    sems = (sem0, sem1)

    pltpu.sync_copy(i_idx_hbm.at[wid], idx_i_v)
    pltpu.sync_copy(j_idx_hbm.at[wid], idx_j_v)

    def fire(k):
        s = k % 2
        pltpu.async_copy(wi_hbm.at[idx_i_v.at[k]], rows_i.at[s], sems[s])
        pltpu.async_copy(wj_hbm.at[idx_j_v.at[k]], rows_j.at[s], sems[s])

    def drain(k):
        s = k % 2
        pltpu.make_async_copy(wi_hbm.at[idx_i_v.at[k]], rows_i.at[s], sems[s]).wait()
        pltpu.make_async_copy(wj_hbm.at[idx_j_v.at[k]], rows_j.at[s], sems[s]).wait()

    lane_iota = lax.broadcasted_iota(jnp.int32, (L,), 0)
    perms = [lane_iota ^ sh for sh in (8, 4, 2, 1)]

    fire(0)
    for k in range(NCHUNK):
        if k + 1 < NCHUNK:
            fire(k + 1)
        drain(k)
        s = k % 2

        def blk_body(bi, carry):
            out_vec = jnp.zeros((L,), jnp.float32)
            for rr in range(L):
                r = bi * L + rr
                acc = rows_i[s, r, pl.ds(0, L)] * rows_j[s, r, pl.ds(0, L)]
                for c in range(1, D // L):
                    acc = acc + (rows_i[s, r, pl.ds(c * L, L)]
                                 * rows_j[s, r, pl.ds(c * L, L)])
                for p in perms:
                    acc = acc + acc.at[p].get(mode="promise_in_bounds")
                out_vec = jnp.where(lane_iota == rr, acc, out_vec)
            out_v[pl.ds(k * CHUNK + bi * L, L)] = out_vec
            return carry

        lax.fori_loop(0, BLK, blk_body, 0)

    pltpu.sync_copy(out_v, out_hbm.at[pl.ds(base, B_PER_W)])


@jax.jit
def _run(i_idx, j_idx, wi, wj):
    mesh = plsc.VectorSubcoreMesh(core_axis_name="c", subcore_axis_name="s")
    kern = functools.partial(
        pl.kernel,
        out_type=jax.ShapeDtypeStruct((B,), jnp.float32),
        mesh=mesh,
        compiler_params=pltpu.CompilerParams(use_tc_tiling_on_sc=False),
        scratch_types=[
            pltpu.VMEM((NCHUNK, CHUNK), jnp.int32),
            pltpu.VMEM((NCHUNK, CHUNK), jnp.int32),
            pltpu.VMEM((2, CHUNK, D), jnp.float32),
            pltpu.VMEM((2, CHUNK, D), jnp.float32),
            pltpu.VMEM((B_PER_W,), jnp.float32),
            pltpu.SemaphoreType.DMA,
            pltpu.SemaphoreType.DMA,
        ],
    )(_sc_kernel)
    return kern(i_idx, j_idx, wi, wj)


def kernel(i_indices, j_indices, wi, wj):
    i_r = i_indices.reshape(NW, NCHUNK, CHUNK)
    j_r = j_indices.reshape(NW, NCHUNK, CHUNK)
    return _run(i_r, j_r, wi, wj)


# speedup vs baseline: 1.0048x; 1.0048x over previous
import functools

import jax
import jax.numpy as jnp
from jax import lax
from jax.experimental import pallas as pl
from jax.experimental.pallas import tpu as pltpu
from jax.experimental.pallas import tpu_sc as plsc

D = 64
B = 16384
NC = 2
NS = 16
NW = NC * NS
B_PER_W = B // NW
CHUNK = 128
NCHUNK = B_PER_W // CHUNK
L = 16
BLK = CHUNK // L


def _sc_kernel(i_idx_hbm, j_idx_hbm, wi_hbm, wj_hbm, out_hbm,
               idx_i_v, idx_j_v, rows_i, rows_j, out_v, sem0, sem1):
    wid = lax.axis_index("s") * NC + lax.axis_index("c")
    base = wid * B_PER_W
    sems = (sem0, sem1)

    pltpu.sync_copy(i_idx_hbm.at[wid], idx_i_v)
    pltpu.sync_copy(j_idx_hbm.at[wid], idx_j_v)

    def fire(k):
        s = k % 2
        pltpu.async_copy(wi_hbm.at[idx_i_v.at[k]], rows_i.at[s], sems[s])
        pltpu.async_copy(wj_hbm.at[idx_j_v.at[k]], rows_j.at[s], sems[s])

    def drain(k):
        s = k % 2
        pltpu.make_async_copy(wi_hbm.at[idx_i_v.at[k]], rows_i.at[s], sems[s]).wait()
        pltpu.make_async_copy(wj_hbm.at[idx_j_v.at[k]], rows_j.at[s], sems[s]).wait()

    lane_iota = lax.broadcasted_iota(jnp.int32, (L,), 0)
    perms = [lane_iota ^ sh for sh in (8, 4, 2, 1)]

    fire(0)
    for k in range(NCHUNK):
        if k + 1 < NCHUNK:
            fire(k + 1)
        drain(k)
        s = k % 2

        def blk_body(bi, carry):
            out_vec = jnp.zeros((L,), jnp.float32)
            for rr in range(L):
                r = bi * L + rr
                acc = rows_i[s, r, pl.ds(0, L)] * rows_j[s, r, pl.ds(0, L)]
                for c in range(1, D // L):
                    acc = acc + (rows_i[s, r, pl.ds(c * L, L)]
                                 * rows_j[s, r, pl.ds(c * L, L)])
                for p in perms:
                    acc = acc + acc.at[p].get(mode="promise_in_bounds")
                out_vec = jnp.where(lane_iota == rr, acc, out_vec)
            out_v[pl.ds(k * CHUNK + bi * L, L)] = out_vec
            return carry

        lax.fori_loop(0, BLK, blk_body, 0)

    pltpu.sync_copy(out_v, out_hbm.at[pl.ds(base, B_PER_W)])


def _pad128(w):
    return jnp.pad(w.T, ((0, 128 - D), (0, 0))).T


@jax.jit
def _run(i_idx, j_idx, wi, wj):
    mesh = plsc.VectorSubcoreMesh(core_axis_name="c", subcore_axis_name="s")
    kern = functools.partial(
        pl.kernel,
        out_type=jax.ShapeDtypeStruct((B,), jnp.float32),
        mesh=mesh,
        compiler_params=pltpu.CompilerParams(use_tc_tiling_on_sc=True),
        scratch_types=[
            pltpu.VMEM((NCHUNK, CHUNK), jnp.int32),
            pltpu.VMEM((NCHUNK, CHUNK), jnp.int32),
            pltpu.VMEM((2, CHUNK, 128), jnp.float32),
            pltpu.VMEM((2, CHUNK, 128), jnp.float32),
            pltpu.VMEM((B_PER_W,), jnp.float32),
            pltpu.SemaphoreType.DMA,
            pltpu.SemaphoreType.DMA,
        ],
    )(_sc_kernel)
    return kern(i_idx, j_idx, _pad128(wi), _pad128(wj))


def kernel(i_indices, j_indices, wi, wj):
    i_r = i_indices.reshape(NW, NCHUNK, CHUNK)
    j_r = j_indices.reshape(NW, NCHUNK, CHUNK)
    return _run(i_r, j_r, wi, wj)


# speedup vs baseline: 1.0509x; 1.0458x over previous
import functools

import jax
import jax.numpy as jnp
from jax import lax
from jax.experimental import pallas as pl
from jax.experimental.pallas import tpu as pltpu
from jax.experimental.pallas import tpu_sc as plsc

D = 64
B = 16384
NC = 2
NS = 16
NW = NC * NS
B_PER_W = B // NW
CHUNK = 128
NCHUNK = B_PER_W // CHUNK
L = 16
BLK = CHUNK // L


def _sc_kernel(i_idx_hbm, j_idx_hbm, wi_hbm, wj_hbm, out_hbm,
               idx_i_v, idx_j_v, rows_i, rows_j, out_v, sem0, sem1):
    wid = lax.axis_index("s") * NC + lax.axis_index("c")
    base = wid * B_PER_W
    sems = (sem0, sem1)

    pltpu.sync_copy(i_idx_hbm.at[wid], idx_i_v)
    pltpu.sync_copy(j_idx_hbm.at[wid], idx_j_v)

    def fire(k):
        s = k % 2
        pltpu.async_copy(wi_hbm.at[idx_i_v.at[k]], rows_i.at[s], sems[s])
        pltpu.async_copy(wj_hbm.at[idx_j_v.at[k]], rows_j.at[s], sems[s])

    def drain(k):
        s = k % 2
        pltpu.make_async_copy(wi_hbm.at[idx_i_v.at[k]], rows_i.at[s], sems[s]).wait()
        pltpu.make_async_copy(wj_hbm.at[idx_j_v.at[k]], rows_j.at[s], sems[s]).wait()

    lane_iota = lax.broadcasted_iota(jnp.int32, (L,), 0)
    perms = [lane_iota ^ sh for sh in (8, 4, 2, 1)]

    fire(0)
    for k in range(NCHUNK):
        if k + 1 < NCHUNK:
            fire(k + 1)
        drain(k)
        s = k % 2

        def blk_body(bi, carry):
            out_vec = jnp.zeros((L,), jnp.float32)
            for rr in range(L):
                r = bi * L + rr
                acc = rows_i[s, r, pl.ds(0, L)] * rows_j[s, r, pl.ds(0, L)]
                for c in range(1, D // L):
                    acc = acc + (rows_i[s, r, pl.ds(c * L, L)]
                                 * rows_j[s, r, pl.ds(c * L, L)])
                for p in perms:
                    acc = acc + acc.at[p].get(mode="promise_in_bounds")
                out_vec = jnp.where(lane_iota == rr, acc, out_vec)
            out_v[pl.ds(k * CHUNK + bi * L, L)] = out_vec
            return carry

        lax.fori_loop(0, BLK, blk_body, 0)

    pltpu.sync_copy(out_v, out_hbm.at[pl.ds(base, B_PER_W)])


def _pad128(w):
    zeros = jnp.zeros((w.shape[0], 128), jnp.float32)
    return lax.dynamic_update_slice(zeros, w, (0, 0))


@jax.jit
def _run(i_idx, j_idx, wi, wj):
    mesh = plsc.VectorSubcoreMesh(core_axis_name="c", subcore_axis_name="s")
    kern = functools.partial(
        pl.kernel,
        out_type=jax.ShapeDtypeStruct((B,), jnp.float32),
        mesh=mesh,
        compiler_params=pltpu.CompilerParams(use_tc_tiling_on_sc=True),
        scratch_types=[
            pltpu.VMEM((NCHUNK, CHUNK), jnp.int32),
            pltpu.VMEM((NCHUNK, CHUNK), jnp.int32),
            pltpu.VMEM((2, CHUNK, 128), jnp.float32),
            pltpu.VMEM((2, CHUNK, 128), jnp.float32),
            pltpu.VMEM((B_PER_W,), jnp.float32),
            pltpu.SemaphoreType.DMA,
            pltpu.SemaphoreType.DMA,
        ],
    )(_sc_kernel)
    return kern(i_idx, j_idx, _pad128(wi), _pad128(wj))


def kernel(i_indices, j_indices, wi, wj):
    i_r = i_indices.reshape(NW, NCHUNK, CHUNK)
    j_r = j_indices.reshape(NW, NCHUNK, CHUNK)
    return _run(i_r, j_r, wi, wj)
